# baseline (speedup 1.0000x reference)
import functools
import jax
import jax.numpy as jnp
from jax import lax
from jax.experimental import pallas as pl
from jax.experimental.pallas import tpu as pltpu
from jax.experimental.pallas import tpu_sc as plsc

_N, _L, _C = 16, 256, 1024
_K = 4
_G = 8
_W = _G * _L
_NG = _N // _G

_NT = (((1,), (1,)), ((), ()))
_HI = jax.lax.Precision.HIGHEST

_NC = 2
_NS = 16
_PW = _N * _L // (_NC * _NS)
_V = 16


def _msm_dist_kernel(zb_ref, out_ref, nc_ref, nr_ref):
    c = pl.program_id(0)

    @pl.when(c == 0)
    def _():
        zf = zb_ref[...].astype(jnp.float32)
        n_all = jnp.sum(zf * zf, axis=1, keepdims=True)
        nc_ref[...] = n_all
        for k in range(_NG):
            nr_ref[k:k + 1, :] = jax.lax.dot_general(
                jnp.ones((1, 1), jnp.float32),
                n_all[k * _W:(k + 1) * _W, :], _NT,
                precision=_HI, preferred_element_type=jnp.float32)

    g = jax.lax.dot_general(zb_ref[...], zb_ref[pl.ds(c * _W, _W), :], _NT,
                            preferred_element_type=jnp.float32)
    n_p = nr_ref[pl.ds(c, 1), :]

    mins = [jnp.min(nc_ref[j * _L:(j + 1) * _L, :] - 2.0 * g[j * _L:(j + 1) * _L, :],
                    axis=0, keepdims=True)
            for j in range(_N)]
    m = jnp.concatenate(mins, axis=0)

    row = jax.lax.broadcasted_iota(jnp.int32, (_N, _W), 0)
    img = c * _G + jax.lax.broadcasted_iota(jnp.int32, (_N, _W), 1) // _L
    dist = jnp.sqrt(jnp.maximum(n_p + m, 1e-12))
    out_ref[...] = jnp.where(row == img, jnp.float32(jnp.inf), dist)


def _sc_score_kernel(d_hbm, out_hbm, mblk, obuf):
    wid = lax.axis_index("s") * _NC + lax.axis_index("c")
    base = wid * _PW
    pltpu.sync_copy(d_hbm.at[:, pl.ds(base, _PW)], mblk)
    inf = jnp.full((_V,), jnp.inf, jnp.float32)
    zero = jnp.full((_V,), 0.0, jnp.float32)
    one = jnp.full((_V,), 1.0, jnp.float32)
    for gidx in range(_PW // _V):
        regs = [mblk[j, pl.ds(gidx * _V, _V)] for j in range(_N)]
        acc = zero
        rem = jnp.full((_V,), jnp.float32(_K))
        for _ in range(_K):
            v = regs[0]
            for j in range(1, _N):
                v = jnp.minimum(v, regs[j])
            cnt = zero
            for j in range(_N):
                cnt = cnt + jnp.where(regs[j] == v, one, zero)
            t = jnp.minimum(cnt, rem)
            acc = acc + jnp.where(t > 0.0, t * v, zero)
            rem = rem - t
            regs = [jnp.where(r == v, inf, r) for r in regs]
        obuf[pl.ds(gidx * _V, _V)] = acc * jnp.float32(1.0 / _K)
    pltpu.sync_copy(obuf, out_hbm.at[pl.ds(base, _PW)])


def kernel(Z):
    N, L, C = Z.shape
    zb = Z.reshape(N * L, C).astype(jnp.bfloat16)
    dist = pl.pallas_call(
        _msm_dist_kernel,
        grid=(_NG,),
        in_specs=[pl.BlockSpec((N * L, C), lambda c: (0, 0))],
        out_specs=pl.BlockSpec((_N, _W), lambda c: (0, c)),
        out_shape=jax.ShapeDtypeStruct((_N, N * L), jnp.float32),
        scratch_shapes=[pltpu.VMEM((N * L, 1), jnp.float32),
                        pltpu.VMEM((_NG, _W), jnp.float32)],
    )(zb)

    mesh = plsc.VectorSubcoreMesh(core_axis_name="c", subcore_axis_name="s")
    score = functools.partial(
        pl.kernel, mesh=mesh,
        out_type=jax.ShapeDtypeStruct((N * L,), jnp.float32),
        scratch_types=[pltpu.VMEM((_N, _PW), jnp.float32),
                       pltpu.VMEM((_PW,), jnp.float32)],
    )(_sc_score_kernel)
    out = score(dist)
    return out.reshape(N, L)


# speedup vs baseline: 1.1102x; 1.1102x over previous
import functools
import jax
import jax.numpy as jnp
from jax import lax
from jax.experimental import pallas as pl
from jax.experimental.pallas import tpu as pltpu
from jax.experimental.pallas import tpu_sc as plsc

_N, _L, _C = 16, 256, 1024
_K = 4
_G = 8
_W = _G * _L
_NG = _N // _G

_NT = (((1,), (1,)), ((), ()))
_HI = jax.lax.Precision.HIGHEST

_NC = 2
_NS = 16
_PW = _N * _L // (_NC * _NS)
_V = 16


def _msm_dist_kernel(z_all_ref, out_ref, zb_ref, nc_ref, nr_ref):
    c = pl.program_id(0)

    @pl.when(c == 0)
    def _():
        z_all = z_all_ref[...]
        n_all = jnp.sum(z_all * z_all, axis=1, keepdims=True)
        nc_ref[...] = n_all
        zb_ref[...] = z_all.astype(jnp.bfloat16)
        for k in range(_NG):
            nr_ref[k:k + 1, :] = jax.lax.dot_general(
                jnp.ones((1, 1), jnp.float32),
                n_all[k * _W:(k + 1) * _W, :], _NT,
                precision=_HI, preferred_element_type=jnp.float32)

    g = jax.lax.dot_general(zb_ref[...], zb_ref[pl.ds(c * _W, _W), :], _NT,
                            preferred_element_type=jnp.float32)
    n_p = nr_ref[pl.ds(c, 1), :]

    mins = [jnp.min(nc_ref[j * _L:(j + 1) * _L, :] - 2.0 * g[j * _L:(j + 1) * _L, :],
                    axis=0, keepdims=True)
            for j in range(_N)]
    m = jnp.concatenate(mins, axis=0)

    row = jax.lax.broadcasted_iota(jnp.int32, (_N, _W), 0)
    img = c * _G + jax.lax.broadcasted_iota(jnp.int32, (_N, _W), 1) // _L
    dist = jnp.sqrt(jnp.maximum(n_p + m, 1e-12))
    out_ref[...] = jnp.where(row == img, jnp.float32(jnp.inf), dist)


def _sc_score_kernel(d_hbm, out_hbm, mblk, obuf):
    wid = lax.axis_index("s") * _NC + lax.axis_index("c")
    base = wid * _PW
    pltpu.sync_copy(d_hbm.at[:, pl.ds(base, _PW)], mblk)
    inf = jnp.full((_V,), jnp.inf, jnp.float32)
    zero = jnp.full((_V,), 0.0, jnp.float32)
    one = jnp.full((_V,), 1.0, jnp.float32)
    for gidx in range(_PW // _V):
        regs = [mblk[j, pl.ds(gidx * _V, _V)] for j in range(_N)]
        acc = zero
        rem = jnp.full((_V,), jnp.float32(_K))
        for _ in range(_K):
            v = regs[0]
            for j in range(1, _N):
                v = jnp.minimum(v, regs[j])
            cnt = zero
            for j in range(_N):
                cnt = cnt + jnp.where(regs[j] == v, one, zero)
            t = jnp.minimum(cnt, rem)
            acc = acc + jnp.where(t > 0.0, t * v, zero)
            rem = rem - t
            regs = [jnp.where(r == v, inf, r) for r in regs]
        obuf[pl.ds(gidx * _V, _V)] = acc * jnp.float32(1.0 / _K)
    pltpu.sync_copy(obuf, out_hbm.at[pl.ds(base, _PW)])


def kernel(Z):
    N, L, C = Z.shape
    z_all = Z.reshape(N * L, C)
    dist = pl.pallas_call(
        _msm_dist_kernel,
        grid=(_NG,),
        in_specs=[pl.BlockSpec((N * L, C), lambda c: (0, 0))],
        out_specs=pl.BlockSpec((_N, _W), lambda c: (0, c)),
        out_shape=jax.ShapeDtypeStruct((_N, N * L), jnp.float32),
        scratch_shapes=[pltpu.VMEM((N * L, C), jnp.bfloat16),
                        pltpu.VMEM((N * L, 1), jnp.float32),
                        pltpu.VMEM((_NG, _W), jnp.float32)],
    )(z_all)

    mesh = plsc.VectorSubcoreMesh(core_axis_name="c", subcore_axis_name="s")
    score = functools.partial(
        pl.kernel, mesh=mesh,
        out_type=jax.ShapeDtypeStruct((N * L,), jnp.float32),
        scratch_types=[pltpu.VMEM((_N, _PW), jnp.float32),
                       pltpu.VMEM((_PW,), jnp.float32)],
    )(_sc_score_kernel)
    out = score(dist)
    return out.reshape(N, L)
